# baseline (speedup 1.0000x reference)
import functools

import jax
import jax.numpy as jnp
from jax import lax
from jax.experimental import pallas as pl
from jax.experimental.pallas import tpu as pltpu
from jax.experimental.pallas import tpu_sc as plsc

N = 10000
E = 320000
H = 128
L = 6

NB = 2000
EB = 2000

SC_CORES = 2
SC_SUBCORES = 16
EPW = E // (SC_CORES * SC_SUBCORES)
SB = 400
SC_ITERS = EPW // SB
SBS = 200
SC_ITERS_S = EPW // SBS

_F32 = jnp.float32


def _dot(a, b):
    return jnp.dot(a, b, preferred_element_type=_F32)


def _full_spec(shape):
    nd = len(shape)
    return pl.BlockSpec(shape, lambda i, _nd=nd: (0,) * _nd)


def _row_spec(cols, blk):
    return pl.BlockSpec((blk, cols), lambda i: (i, 0))


def _enc_body(x_ref, wne0_ref, bne0_ref, wne1_ref, bne1_ref, h_ref):
    t = jnp.maximum(_dot(x_ref[...], wne0_ref[...]) + bne0_ref[...], 0.0)
    h_ref[...] = _dot(t, wne1_ref[...]) + bne1_ref[...]


def _enc_call(x, wne0, bne0, wne1, bne1):
    return pl.pallas_call(
        _enc_body,
        grid=(N // NB,),
        in_specs=[_row_spec(x.shape[1], NB), _full_spec(wne0.shape),
                  _full_spec(bne0.shape), _full_spec(wne1.shape),
                  _full_spec(bne1.shape)],
        out_specs=_row_spec(H, NB),
        out_shape=jax.ShapeDtypeStruct((N, H), _F32),
    )(x, wne0, bne0, wne1, bne1)


def _edge_body(ea_ref, g1_ref, g2_ref, wee0_ref, bee0_ref, wee1_ref, bee1_ref,
               we0_ref, be0_ref, we1_ref, be1_ref, m_ref):
    a = jnp.maximum(_dot(ea_ref[...], wee0_ref[...]) + bee0_ref[...], 0.0)
    e = _dot(a, wee1_ref[...]) + bee1_ref[...]
    m_in = jnp.concatenate([g1_ref[...], g2_ref[...], e], axis=1)
    t = jnp.maximum(_dot(m_in, we0_ref[...]) + be0_ref[...], 0.0)
    m_ref[...] = _dot(t, we1_ref[...]) + be1_ref[...]


def _edge_call(ea, g1, g2, wee0, bee0, wee1, bee1, we0, be0, we1, be1):
    return pl.pallas_call(
        _edge_body,
        grid=(E // EB,),
        in_specs=[_row_spec(ea.shape[1], EB), _row_spec(H, EB), _row_spec(H, EB),
                  _full_spec(wee0.shape), _full_spec(bee0.shape),
                  _full_spec(wee1.shape), _full_spec(bee1.shape),
                  _full_spec(we0.shape), _full_spec(be0.shape),
                  _full_spec(we1.shape), _full_spec(be1.shape)],
        out_specs=_row_spec(H, EB),
        out_shape=jax.ShapeDtypeStruct((E, H), _F32),
    )(ea, g1, g2, wee0, bee0, wee1, bee1, we0, be0, we1, be1)


def _node_body(h_ref, agg_ref, wn0_ref, bn0_ref, wn1_ref, bn1_ref,
               hout_ref):
    h = h_ref[...]
    u_in = jnp.concatenate([h, agg_ref[...]], axis=1)
    t = jnp.maximum(_dot(u_in, wn0_ref[...]) + bn0_ref[...], 0.0)
    hout_ref[...] = h + _dot(t, wn1_ref[...]) + bn1_ref[...]


def _node_call(h, agg, wn0, bn0, wn1, bn1):
    return pl.pallas_call(
        _node_body,
        grid=(N // NB,),
        in_specs=[_row_spec(H, NB), _row_spec(H, NB),
                  _full_spec(wn0.shape), _full_spec(bn0.shape),
                  _full_spec(wn1.shape), _full_spec(bn1.shape)],
        out_specs=_row_spec(H, NB),
        out_shape=jax.ShapeDtypeStruct((N, H), _F32),
    )(h, agg, wn0, bn0, wn1, bn1)


def _dec_body(h_ref, wd0_ref, bd0_ref, wd1_ref, bd1_ref, wd2_ref, bd2_ref,
              mask_ref, out_ref):
    z = jnp.maximum(_dot(h_ref[...], wd0_ref[...]) + bd0_ref[...], 0.0)
    z = jnp.maximum(_dot(z, wd1_ref[...]) + bd1_ref[...], 0.0)
    pred = _dot(z, wd2_ref[...]) + bd2_ref[...]
    out_ref[...] = pred * mask_ref[...]


def _dec_call(h, wd0, bd0, wd1, bd1, wd2, bd2, mask):
    return pl.pallas_call(
        _dec_body,
        grid=(N // NB,),
        in_specs=[_row_spec(H, NB), _full_spec(wd0.shape), _full_spec(bd0.shape),
                  _full_spec(wd1.shape), _full_spec(bd1.shape),
                  _full_spec(wd2.shape), _full_spec(bd2.shape),
                  _row_spec(3, NB)],
        out_specs=_row_spec(3, NB),
        out_shape=jax.ShapeDtypeStruct((N, 3), _F32),
    )(h, wd0, bd0, wd1, bd1, wd2, bd2, mask)


def _sc_mesh():
    return plsc.VectorSubcoreMesh(core_axis_name="c", subcore_axis_name="s")


def _sc_gather(h, src, dst):
    @functools.partial(
        pl.kernel,
        out_type=(jax.ShapeDtypeStruct((E, H), _F32),
                  jax.ShapeDtypeStruct((E, H), _F32)),
        mesh=_sc_mesh(),
        scratch_types=[pltpu.VMEM((SB,), jnp.int32),
                       pltpu.VMEM((SB, H), _F32),
                       pltpu.SemaphoreType.DMA],
    )
    def gk(h_hbm, src_hbm, dst_hbm, g1_hbm, g2_hbm, idx_v, rows_v, sem):
        c = lax.axis_index("c")
        s = lax.axis_index("s")
        base = (c * SC_SUBCORES + s) * EPW

        @pl.loop(0, SC_ITERS)
        def _(i):
            off = base + i * SB
            pltpu.sync_copy(src_hbm.at[pl.ds(off, SB)], idx_v)
            pltpu.async_copy(h_hbm.at[idx_v], rows_v, sem).wait()
            pltpu.sync_copy(rows_v, g1_hbm.at[pl.ds(off, SB)])
            pltpu.sync_copy(dst_hbm.at[pl.ds(off, SB)], idx_v)
            pltpu.async_copy(h_hbm.at[idx_v], rows_v, sem).wait()
            pltpu.sync_copy(rows_v, g2_hbm.at[pl.ds(off, SB)])

    return gk(h, src, dst)


def _seg_body(dst_ref, m_ref, out_ref, acc_ref):
    i = pl.program_id(0)

    @pl.when(i == 0)
    def _():
        acc_ref[...] = jnp.zeros_like(acc_ref)

    def body(j, carry):
        idx = dst_ref[0, 0, j]
        acc_ref[pl.ds(idx, 1), :] += m_ref[pl.ds(j, 1), :]
        return carry

    lax.fori_loop(0, EB, body, 0)

    @pl.when(i == E // EB - 1)
    def _():
        out_ref[...] = acc_ref[...]


def _seg_call(m, dst3):
    return pl.pallas_call(
        _seg_body,
        grid=(E // EB,),
        in_specs=[pl.BlockSpec((1, 1, EB), lambda i: (i, 0, 0),
                               memory_space=pltpu.SMEM),
                  _row_spec(H, EB)],
        out_specs=_full_spec((N, H)),
        out_shape=jax.ShapeDtypeStruct((N, H), _F32),
        scratch_shapes=[pltpu.VMEM((N, H), _F32)],
    )(dst3, m)


def kernel(x, edge_attr, edge_index, bc_disp, bc_rot, Wne0, bne0, Wne1, bne1,
           Wee0, bee0, Wee1, bee1, We0, be0, We1, be1, Wn0, bn0, Wn1, bn1,
           Wd0, bd0, Wd1, bd1, Wd2, bd2):
    src = edge_index[0]
    dst = edge_index[1]
    mask = 1.0 - jnp.concatenate([bc_disp, bc_rot], axis=1)

    r2 = lambda b: b.reshape(1, -1)
    dst3 = dst.reshape(E // EB, 1, EB)

    h = _enc_call(x, Wne0, r2(bne0), Wne1, r2(bne1))

    def step(h, ws):
        we0, be0_l, we1, be1_l, wn0, bn0_l, wn1, bn1_l = ws
        g1, g2 = _sc_gather(h, src, dst)
        m = _edge_call(edge_attr, g1, g2, Wee0, r2(bee0), Wee1, r2(bee1),
                       we0, be0_l, we1, be1_l)
        agg = _seg_call(m, dst3)
        h2 = _node_call(h, agg, wn0, bn0_l, wn1, bn1_l)
        return h2, None

    ws = (We0, be0.reshape(L, 1, H), We1, be1.reshape(L, 1, H),
          Wn0, bn0.reshape(L, 1, H), Wn1, bn1.reshape(L, 1, H))
    h, _ = lax.scan(step, h, ws)

    return _dec_call(h, Wd0, r2(bd0), Wd1, r2(bd1), Wd2, r2(bd2), mask)
